# baseline (speedup 1.0000x reference)
import functools

import jax
import jax.numpy as jnp
from jax import lax
from jax.experimental import pallas as pl
from jax.experimental.pallas import tpu as pltpu
from jax.experimental.pallas import tpu_sc as plsc

N = 10000
E = 320000
F = 128
HID = 64
C = 40
K = 10
CP = 64
NW = 32
N2 = 2 * N
NV = N // 16
CE = 16000
NCHUNK = E // CE


def _mlp_body(x_ref, w1_ref, b1_ref, w2_ref, b2_ref, o_ref):
    h1 = jnp.maximum(jnp.dot(x_ref[...], w1_ref[...],
                             preferred_element_type=jnp.float32) + b1_ref[...], 0.0)
    o_ref[...] = jnp.dot(h1, w2_ref[...],
                         preferred_element_type=jnp.float32) + b2_ref[...]


def _lsm_body(u_ref, o_ref):
    u = u_ref[...]
    ids = lax.broadcasted_iota(jnp.int32, u.shape, 1)
    m = jnp.where(ids < C, u, jnp.float32(-1e30))
    mx = jnp.max(m, axis=1, keepdims=True)
    e = jnp.exp(m - mx)
    s = jnp.sum(e, axis=1, keepdims=True)
    o_ref[...] = (u - mx - jnp.log(s))[:, :C]


def _sc_body(h_hbm, ep_hbm, tb_hbm, out_hbm,
             V0, V1, T0, T1, H0, H1, dsv, eb0, eb1, tv, sem0, sem1):
    wid = lax.axis_index("s") * 2 + lax.axis_index("c")
    ones = jnp.ones((16,), jnp.float32)
    eb = (eb0, eb1)
    sems = (sem0, sem1)

    pltpu.sync_copy(tb_hbm, tv)
    pltpu.sync_copy(h_hbm.at[2 * wid], V0)
    pltpu.sync_copy(h_hbm.at[2 * wid + 1], V1)

    def _issue(ch, b):
        pltpu.async_copy(ep_hbm.at[pl.ds(ch * CE, CE)], eb[b], sems[b])

    def _wait(b):
        pltpu.make_async_copy(ep_hbm.at[pl.ds(0, CE)], eb[b], sems[b]).wait()

    def _edge_pass(grp_fn):
        def _process(b):
            @plsc.parallel_loop(0, CE // 16, unroll=16)
            def _grp(g):
                grp_fn(eb[b], g)

        _issue(0, 0)

        def _pair(p, c):
            ch0 = p * 2
            _issue(ch0 + 1, 1)
            _wait(0)
            _process(0)

            @pl.when(p + 1 < NCHUNK // 2)
            def _():
                _issue(ch0 + 2, 0)
            _wait(1)
            _process(1)
            return c
        lax.fori_loop(0, NCHUNK // 2, _pair, 0)

    @plsc.parallel_loop(0, NV, unroll=8)
    def _init_t(i):
        dsv[pl.ds(i * 16, 16)] = ones

    def _deg_grp(ebuf, g):
        ev = ebuf[pl.ds(g * 16, 16)]
        cv = lax.shift_right_logical(ev, 16)
        plsc.addupdate_scatter(dsv, [cv], ones)
    with jax.named_scope("deg_pass"):
        _edge_pass(_deg_grp)

    t0 = tv[pl.ds(0, 16)]

    @plsc.parallel_loop(0, NV, unroll=4)
    def _node_init(i):
        sl = pl.ds(i * 16, 16)
        d = dsv[sl]
        di = plsc.bitcast(d, jnp.int32)
        y = plsc.bitcast(jnp.int32(0x5F3759DF) - lax.shift_right_logical(di, 1),
                         jnp.float32)
        for _ in range(4):
            y = y * (jnp.float32(1.5) - jnp.float32(0.5) * d * y * y)
        dsv[sl] = y
        q0 = y * V0[sl]
        V0[sl] = q0
        T0[sl] = q0
        H0[sl] = t0 * q0
        q1 = y * V1[sl]
        V1[sl] = q1
        T1[sl] = q1
        H1[sl] = t0 * q1

    def _edge_grp(ebuf, g):
        ev = ebuf[pl.ds(g * 16, 16)]
        rv = jnp.bitwise_and(ev, jnp.int32(0xFFFF))
        cv = lax.shift_right_logical(ev, 16)
        g0 = plsc.load_gather(V0, [rv])
        g1 = plsc.load_gather(V1, [rv])
        plsc.addupdate_scatter(T0, [cv], g0)
        plsc.addupdate_scatter(T1, [cv], g1)

    for k in range(K):
        with jax.named_scope("edge_hop"):
            _edge_pass(_edge_grp)
        tk = tv[pl.ds(16 * (k + 1), 16)]

        @plsc.parallel_loop(0, NV, unroll=8)
        def _scale(i):
            sl = pl.ds(i * 16, 16)
            s = dsv[sl]
            ss = s * s
            q0 = ss * T0[sl]
            V0[sl] = q0
            T0[sl] = q0
            H0[sl] = H0[sl] + tk * q0
            q1 = ss * T1[sl]
            V1[sl] = q1
            T1[sl] = q1
            H1[sl] = H1[sl] + tk * q1

    @plsc.parallel_loop(0, NV, unroll=8)
    def _fin(i):
        sl = pl.ds(i * 16, 16)
        s = dsv[sl]
        V0[sl] = H0[sl] / s
        V1[sl] = H1[sl] / s
    pltpu.sync_copy(V0, out_hbm.at[2 * wid])
    pltpu.sync_copy(V1, out_hbm.at[2 * wid + 1])


_sc_prop = functools.partial(
    pl.kernel,
    out_type=jax.ShapeDtypeStruct((CP, N), jnp.float32),
    mesh=plsc.VectorSubcoreMesh(core_axis_name="c", subcore_axis_name="s"),
    compiler_params=pltpu.CompilerParams(needs_layout_passes=False,
                                         disable_bounds_checks=True),
    scratch_types=[
        pltpu.VMEM((N,), jnp.float32),
        pltpu.VMEM((N,), jnp.float32),
        pltpu.VMEM((N,), jnp.float32),
        pltpu.VMEM((N,), jnp.float32),
        pltpu.VMEM((N,), jnp.float32),
        pltpu.VMEM((N,), jnp.float32),
        pltpu.VMEM((N,), jnp.float32),
        pltpu.VMEM((CE,), jnp.int32),
        pltpu.VMEM((CE,), jnp.int32),
        pltpu.VMEM((16 * (K + 1),), jnp.float32),
        pltpu.SemaphoreType.DMA,
        pltpu.SemaphoreType.DMA,
    ],
)(_sc_body)


def kernel(x, edge_index, W1, b1, W2, b2, temp):
    w2p = jnp.pad(W2, ((0, 0), (0, CP - C)))
    b2p = jnp.pad(b2, (0, CP - C)).reshape(1, CP)
    h_pad = pl.pallas_call(
        _mlp_body,
        out_shape=jax.ShapeDtypeStruct((N, CP), jnp.float32),
    )(x, W1, b1.reshape(1, HID), w2p, b2p)

    h64 = h_pad.transpose(1, 0)
    tb = jnp.broadcast_to(temp.reshape(K + 1, 1), (K + 1, 16)).reshape(-1)
    ei = edge_index.astype(jnp.int32)
    ep = jnp.bitwise_or(ei[0], jnp.left_shift(ei[1], 16))
    u64 = _sc_prop(h64, ep, tb)
    u = u64.transpose(1, 0)

    return pl.pallas_call(
        _lsm_body,
        out_shape=jax.ShapeDtypeStruct((N, C), jnp.float32),
    )(u)


# speedup vs baseline: 1.0368x; 1.0368x over previous
import functools

import jax
import jax.numpy as jnp
from jax import lax
from jax.experimental import pallas as pl
from jax.experimental.pallas import tpu as pltpu
from jax.experimental.pallas import tpu_sc as plsc

N = 10000
E = 320000
F = 128
HID = 64
C = 40
K = 10
CP = 64
NW = 32
N2 = 2 * N
NV = N // 16
CE = 16000
NCHUNK = E // CE
ET = E // NW


def _mlp_body(x_ref, w1_ref, b1_ref, w2_ref, b2_ref, dp_ref, o_ref, d_ref):
    h1 = jnp.maximum(jnp.dot(x_ref[...], w1_ref[...],
                             preferred_element_type=jnp.float32) + b1_ref[...], 0.0)
    o_ref[...] = jnp.dot(h1, w2_ref[...],
                         preferred_element_type=jnp.float32) + b2_ref[...]
    deg = jnp.sum(dp_ref[...], axis=0, keepdims=True) + 1.0
    d_ref[...] = lax.rsqrt(deg)


def _sc_deg_body(ep_hbm, out_hbm, cnt, eb):
    wid = lax.axis_index("s") * 2 + lax.axis_index("c")
    ones = jnp.ones((16,), jnp.float32)

    @plsc.parallel_loop(0, NV, unroll=8)
    def _init(i):
        cnt[pl.ds(i * 16, 16)] = jnp.zeros((16,), jnp.float32)

    pltpu.sync_copy(ep_hbm.at[pl.ds(wid * ET, ET)], eb)

    @plsc.parallel_loop(0, ET // 16, unroll=16)
    def _grp(g):
        ev = eb[pl.ds(g * 16, 16)]
        cv = lax.shift_right_logical(ev, 16)
        plsc.addupdate_scatter(cnt, [cv], ones)

    pltpu.sync_copy(cnt, out_hbm.at[wid])


_sc_deg = functools.partial(
    pl.kernel,
    out_type=jax.ShapeDtypeStruct((NW, N), jnp.float32),
    mesh=plsc.VectorSubcoreMesh(core_axis_name="c", subcore_axis_name="s"),
    compiler_params=pltpu.CompilerParams(needs_layout_passes=False,
                                         disable_bounds_checks=True),
    scratch_types=[
        pltpu.VMEM((N,), jnp.float32),
        pltpu.VMEM((ET,), jnp.int32),
    ],
)(_sc_deg_body)


def _lsm_body(u_ref, o_ref):
    u = u_ref[...]
    ids = lax.broadcasted_iota(jnp.int32, u.shape, 1)
    m = jnp.where(ids < C, u, jnp.float32(-1e30))
    mx = jnp.max(m, axis=1, keepdims=True)
    e = jnp.exp(m - mx)
    s = jnp.sum(e, axis=1, keepdims=True)
    o_ref[...] = (u - mx - jnp.log(s))[:, :C]


def _sc_body(h_hbm, ep_hbm, tb_hbm, d_hbm, out_hbm,
             V0, V1, T0, T1, H0, H1, dsv, eb0, eb1, tv, sem0, sem1):
    wid = lax.axis_index("s") * 2 + lax.axis_index("c")
    eb = (eb0, eb1)
    sems = (sem0, sem1)

    pltpu.sync_copy(tb_hbm, tv)
    pltpu.sync_copy(d_hbm, dsv)
    pltpu.sync_copy(h_hbm.at[2 * wid], V0)
    pltpu.sync_copy(h_hbm.at[2 * wid + 1], V1)

    def _issue(ch, b):
        pltpu.async_copy(ep_hbm.at[pl.ds(ch * CE, CE)], eb[b], sems[b])

    def _wait(b):
        pltpu.make_async_copy(ep_hbm.at[pl.ds(0, CE)], eb[b], sems[b]).wait()

    def _edge_pass(grp_fn):
        def _process(b):
            @plsc.parallel_loop(0, CE // 16, unroll=16)
            def _grp(g):
                grp_fn(eb[b], g)

        _issue(0, 0)

        def _pair(p, c):
            ch0 = p * 2
            _issue(ch0 + 1, 1)
            _wait(0)
            _process(0)

            @pl.when(p + 1 < NCHUNK // 2)
            def _():
                _issue(ch0 + 2, 0)
            _wait(1)
            _process(1)
            return c
        lax.fori_loop(0, NCHUNK // 2, _pair, 0)

    t0 = tv[pl.ds(0, 16)]

    @plsc.parallel_loop(0, NV, unroll=4)
    def _node_init(i):
        sl = pl.ds(i * 16, 16)
        y = dsv[sl]
        q0 = y * V0[sl]
        V0[sl] = q0
        T0[sl] = q0
        H0[sl] = t0 * q0
        q1 = y * V1[sl]
        V1[sl] = q1
        T1[sl] = q1
        H1[sl] = t0 * q1

    def _edge_grp(ebuf, g):
        ev = ebuf[pl.ds(g * 16, 16)]
        rv = jnp.bitwise_and(ev, jnp.int32(0xFFFF))
        cv = lax.shift_right_logical(ev, 16)
        g0 = plsc.load_gather(V0, [rv])
        g1 = plsc.load_gather(V1, [rv])
        plsc.addupdate_scatter(T0, [cv], g0)
        plsc.addupdate_scatter(T1, [cv], g1)

    for k in range(K):
        with jax.named_scope("edge_hop"):
            _edge_pass(_edge_grp)
        tk = tv[pl.ds(16 * (k + 1), 16)]

        @plsc.parallel_loop(0, NV, unroll=8)
        def _scale(i):
            sl = pl.ds(i * 16, 16)
            s = dsv[sl]
            ss = s * s
            q0 = ss * T0[sl]
            V0[sl] = q0
            T0[sl] = q0
            H0[sl] = H0[sl] + tk * q0
            q1 = ss * T1[sl]
            V1[sl] = q1
            T1[sl] = q1
            H1[sl] = H1[sl] + tk * q1

    @plsc.parallel_loop(0, NV, unroll=8)
    def _fin(i):
        sl = pl.ds(i * 16, 16)
        s = dsv[sl]
        V0[sl] = H0[sl] / s
        V1[sl] = H1[sl] / s
    pltpu.sync_copy(V0, out_hbm.at[2 * wid])
    pltpu.sync_copy(V1, out_hbm.at[2 * wid + 1])


_sc_prop = functools.partial(
    pl.kernel,
    out_type=jax.ShapeDtypeStruct((CP, N), jnp.float32),
    mesh=plsc.VectorSubcoreMesh(core_axis_name="c", subcore_axis_name="s"),
    compiler_params=pltpu.CompilerParams(needs_layout_passes=False,
                                         disable_bounds_checks=True),
    scratch_types=[
        pltpu.VMEM((N,), jnp.float32),
        pltpu.VMEM((N,), jnp.float32),
        pltpu.VMEM((N,), jnp.float32),
        pltpu.VMEM((N,), jnp.float32),
        pltpu.VMEM((N,), jnp.float32),
        pltpu.VMEM((N,), jnp.float32),
        pltpu.VMEM((N,), jnp.float32),
        pltpu.VMEM((CE,), jnp.int32),
        pltpu.VMEM((CE,), jnp.int32),
        pltpu.VMEM((16 * (K + 1),), jnp.float32),
        pltpu.SemaphoreType.DMA,
        pltpu.SemaphoreType.DMA,
    ],
)(_sc_body)


def kernel(x, edge_index, W1, b1, W2, b2, temp):
    w2p = jnp.pad(W2, ((0, 0), (0, CP - C)))
    b2p = jnp.pad(b2, (0, CP - C)).reshape(1, CP)
    ei = edge_index.astype(jnp.int32)
    ep = jnp.bitwise_or(ei[0], jnp.left_shift(ei[1], 16))

    degp = _sc_deg(ep)
    h_pad, dsv = pl.pallas_call(
        _mlp_body,
        out_shape=(jax.ShapeDtypeStruct((N, CP), jnp.float32),
                   jax.ShapeDtypeStruct((1, N), jnp.float32)),
    )(x, W1, b1.reshape(1, HID), w2p, b2p, degp)

    h64 = h_pad.transpose(1, 0)
    tb = jnp.broadcast_to(temp.reshape(K + 1, 1), (K + 1, 16)).reshape(-1)
    u64 = _sc_prop(h64, ep, tb, dsv.reshape(N))
    u = u64.transpose(1, 0)

    return pl.pallas_call(
        _lsm_body,
        out_shape=jax.ShapeDtypeStruct((N, C), jnp.float32),
    )(u)


# speedup vs baseline: 1.0452x; 1.0081x over previous
import functools

import jax
import jax.numpy as jnp
from jax import lax
from jax.experimental import pallas as pl
from jax.experimental.pallas import tpu as pltpu
from jax.experimental.pallas import tpu_sc as plsc

N = 10000
E = 320000
F = 128
HID = 64
C = 40
K = 10
CP = 64
NW = 32
N2 = 2 * N
NV = N // 16
CE = 16000
NCHUNK = E // CE
ET = E // NW


def _mlp_body(x_ref, w1_ref, b1_ref, w2_ref, b2_ref, dp_ref, o_ref, d_ref):
    h1 = jnp.maximum(jnp.dot(x_ref[...], w1_ref[...],
                             preferred_element_type=jnp.float32) + b1_ref[...], 0.0)
    o_ref[...] = jnp.dot(h1, w2_ref[...],
                         preferred_element_type=jnp.float32) + b2_ref[...]
    deg = jnp.sum(dp_ref[...], axis=0, keepdims=True) + 1.0
    d_ref[...] = lax.rsqrt(deg)


def _sc_deg_body(ep_hbm, out_hbm, cnt, eb):
    wid = lax.axis_index("s") * 2 + lax.axis_index("c")
    ones = jnp.ones((16,), jnp.float32)

    @plsc.parallel_loop(0, NV, unroll=8)
    def _init(i):
        cnt[pl.ds(i * 16, 16)] = jnp.zeros((16,), jnp.float32)

    pltpu.sync_copy(ep_hbm.at[pl.ds(wid * ET, ET)], eb)

    @plsc.parallel_loop(0, ET // 16, unroll=16)
    def _grp(g):
        ev = eb[pl.ds(g * 16, 16)]
        cv = lax.shift_right_logical(ev, 16)
        plsc.addupdate_scatter(cnt, [cv], ones)

    pltpu.sync_copy(cnt, out_hbm.at[wid])


_sc_deg = functools.partial(
    pl.kernel,
    out_type=jax.ShapeDtypeStruct((NW, N), jnp.float32),
    mesh=plsc.VectorSubcoreMesh(core_axis_name="c", subcore_axis_name="s"),
    compiler_params=pltpu.CompilerParams(needs_layout_passes=False,
                                         disable_bounds_checks=True),
    scratch_types=[
        pltpu.VMEM((N,), jnp.float32),
        pltpu.VMEM((ET,), jnp.int32),
    ],
)(_sc_deg_body)


def _lsm_body(u_ref, o_ref):
    u = u_ref[...]
    ids = lax.broadcasted_iota(jnp.int32, u.shape, 1)
    m = jnp.where(ids < C, u, jnp.float32(-1e30))
    mx = jnp.max(m, axis=1, keepdims=True)
    e = jnp.exp(m - mx)
    s = jnp.sum(e, axis=1, keepdims=True)
    o_ref[...] = (u - mx - jnp.log(s))[:, :C]


def _sc_body(h_hbm, ep_hbm, tb_hbm, d_hbm, out_hbm,
             V0, V1, T0, T1, H0, H1, dsv, eb0, eb1, tv, sem0, sem1):
    wid = lax.axis_index("s") * 2 + lax.axis_index("c")
    eb = (eb0, eb1)
    sems = (sem0, sem1)

    pltpu.sync_copy(tb_hbm, tv)
    pltpu.sync_copy(d_hbm, dsv)
    pltpu.sync_copy(h_hbm.at[2 * wid], V0)
    pltpu.sync_copy(h_hbm.at[2 * wid + 1], V1)

    def _issue(ch, b):
        pltpu.async_copy(ep_hbm.at[pl.ds(ch * CE, CE)], eb[b], sems[b])

    def _wait(b):
        pltpu.make_async_copy(ep_hbm.at[pl.ds(0, CE)], eb[b], sems[b]).wait()

    def _edge_pass(grp_fn):
        def _process(b):
            @plsc.parallel_loop(0, CE // 16, unroll=32)
            def _grp(g):
                grp_fn(eb[b], g)

        _issue(0, 0)

        def _pair(p, c):
            ch0 = p * 2
            _issue(ch0 + 1, 1)
            _wait(0)
            _process(0)

            @pl.when(p + 1 < NCHUNK // 2)
            def _():
                _issue(ch0 + 2, 0)
            _wait(1)
            _process(1)
            return c
        lax.fori_loop(0, NCHUNK // 2, _pair, 0)

    t0 = tv[pl.ds(0, 16)]

    @plsc.parallel_loop(0, NV, unroll=4)
    def _node_init(i):
        sl = pl.ds(i * 16, 16)
        y = dsv[sl]
        q0 = y * V0[sl]
        V0[sl] = q0
        T0[sl] = q0
        H0[sl] = t0 * q0
        q1 = y * V1[sl]
        V1[sl] = q1
        T1[sl] = q1
        H1[sl] = t0 * q1

    def _edge_grp(ebuf, g):
        ev = ebuf[pl.ds(g * 16, 16)]
        rv = jnp.bitwise_and(ev, jnp.int32(0xFFFF))
        cv = lax.shift_right_logical(ev, 16)
        g0 = plsc.load_gather(V0, [rv])
        g1 = plsc.load_gather(V1, [rv])
        plsc.addupdate_scatter(T0, [cv], g0)
        plsc.addupdate_scatter(T1, [cv], g1)

    for k in range(K):
        with jax.named_scope("edge_hop"):
            _edge_pass(_edge_grp)
        tk = tv[pl.ds(16 * (k + 1), 16)]

        @plsc.parallel_loop(0, NV, unroll=8)
        def _scale(i):
            sl = pl.ds(i * 16, 16)
            s = dsv[sl]
            ss = s * s
            q0 = ss * T0[sl]
            V0[sl] = q0
            T0[sl] = q0
            H0[sl] = H0[sl] + tk * q0
            q1 = ss * T1[sl]
            V1[sl] = q1
            T1[sl] = q1
            H1[sl] = H1[sl] + tk * q1

    @plsc.parallel_loop(0, NV, unroll=8)
    def _fin(i):
        sl = pl.ds(i * 16, 16)
        s = dsv[sl]
        V0[sl] = H0[sl] / s
        V1[sl] = H1[sl] / s
    pltpu.sync_copy(V0, out_hbm.at[2 * wid])
    pltpu.sync_copy(V1, out_hbm.at[2 * wid + 1])


_sc_prop = functools.partial(
    pl.kernel,
    out_type=jax.ShapeDtypeStruct((CP, N), jnp.float32),
    mesh=plsc.VectorSubcoreMesh(core_axis_name="c", subcore_axis_name="s"),
    compiler_params=pltpu.CompilerParams(needs_layout_passes=False,
                                         disable_bounds_checks=True),
    scratch_types=[
        pltpu.VMEM((N,), jnp.float32),
        pltpu.VMEM((N,), jnp.float32),
        pltpu.VMEM((N,), jnp.float32),
        pltpu.VMEM((N,), jnp.float32),
        pltpu.VMEM((N,), jnp.float32),
        pltpu.VMEM((N,), jnp.float32),
        pltpu.VMEM((N,), jnp.float32),
        pltpu.VMEM((CE,), jnp.int32),
        pltpu.VMEM((CE,), jnp.int32),
        pltpu.VMEM((16 * (K + 1),), jnp.float32),
        pltpu.SemaphoreType.DMA,
        pltpu.SemaphoreType.DMA,
    ],
)(_sc_body)


def kernel(x, edge_index, W1, b1, W2, b2, temp):
    w2p = jnp.pad(W2, ((0, 0), (0, CP - C)))
    b2p = jnp.pad(b2, (0, CP - C)).reshape(1, CP)
    ei = edge_index.astype(jnp.int32)
    ep = jnp.bitwise_or(ei[0], jnp.left_shift(ei[1], 16))

    degp = _sc_deg(ep)
    h_pad, dsv = pl.pallas_call(
        _mlp_body,
        out_shape=(jax.ShapeDtypeStruct((N, CP), jnp.float32),
                   jax.ShapeDtypeStruct((1, N), jnp.float32)),
    )(x, W1, b1.reshape(1, HID), w2p, b2p, degp)

    h64 = h_pad.transpose(1, 0)
    tb = jnp.broadcast_to(temp.reshape(K + 1, 1), (K + 1, 16)).reshape(-1)
    u64 = _sc_prop(h64, ep, tb, dsv.reshape(N))
    u = u64.transpose(1, 0)

    return pl.pallas_call(
        _lsm_body,
        out_shape=jax.ShapeDtypeStruct((N, C), jnp.float32),
    )(u)


# speedup vs baseline: 1.0504x; 1.0050x over previous
import functools

import jax
import jax.numpy as jnp
from jax import lax
from jax.experimental import pallas as pl
from jax.experimental.pallas import tpu as pltpu
from jax.experimental.pallas import tpu_sc as plsc

N = 10000
E = 320000
F = 128
HID = 64
C = 40
K = 10
CP = 64
NW = 32
N2 = 2 * N
NV = N // 16
CE = 20000
NCHUNK = E // CE
ET = E // NW


def _mlp_body(x_ref, w1_ref, b1_ref, w2_ref, b2_ref, dp_ref, o_ref, d_ref):
    h1 = jnp.maximum(jnp.dot(x_ref[...], w1_ref[...],
                             preferred_element_type=jnp.float32) + b1_ref[...], 0.0)
    o_ref[...] = jnp.dot(h1, w2_ref[...],
                         preferred_element_type=jnp.float32) + b2_ref[...]
    deg = jnp.sum(dp_ref[...], axis=0, keepdims=True) + 1.0
    d_ref[...] = lax.rsqrt(deg)


def _sc_deg_body(ep_hbm, out_hbm, cnt, eb):
    wid = lax.axis_index("s") * 2 + lax.axis_index("c")
    ones = jnp.ones((16,), jnp.float32)

    @plsc.parallel_loop(0, NV, unroll=8)
    def _init(i):
        cnt[pl.ds(i * 16, 16)] = jnp.zeros((16,), jnp.float32)

    pltpu.sync_copy(ep_hbm.at[pl.ds(wid * ET, ET)], eb)

    @plsc.parallel_loop(0, ET // 16, unroll=16)
    def _grp(g):
        ev = eb[pl.ds(g * 16, 16)]
        cv = lax.shift_right_logical(ev, 16)
        plsc.addupdate_scatter(cnt, [cv], ones)

    pltpu.sync_copy(cnt, out_hbm.at[wid])


_sc_deg = functools.partial(
    pl.kernel,
    out_type=jax.ShapeDtypeStruct((NW, N), jnp.float32),
    mesh=plsc.VectorSubcoreMesh(core_axis_name="c", subcore_axis_name="s"),
    compiler_params=pltpu.CompilerParams(needs_layout_passes=False,
                                         disable_bounds_checks=True),
    scratch_types=[
        pltpu.VMEM((N,), jnp.float32),
        pltpu.VMEM((ET,), jnp.int32),
    ],
)(_sc_deg_body)


def _lsm_body(u_ref, o_ref):
    u = u_ref[...]
    ids = lax.broadcasted_iota(jnp.int32, u.shape, 1)
    m = jnp.where(ids < C, u, jnp.float32(-1e30))
    mx = jnp.max(m, axis=1, keepdims=True)
    e = jnp.exp(m - mx)
    s = jnp.sum(e, axis=1, keepdims=True)
    o_ref[...] = (u - mx - jnp.log(s))[:, :C]


def _sc_body(h_hbm, ep_hbm, tb_hbm, d_hbm, out_hbm,
             V0, V1, T0, T1, H0, H1, dsv, eb0, eb1, tv, sem0, sem1):
    wid = lax.axis_index("s") * 2 + lax.axis_index("c")
    eb = (eb0, eb1)
    sems = (sem0, sem1)

    pltpu.sync_copy(tb_hbm, tv)
    pltpu.sync_copy(d_hbm, dsv)
    pltpu.sync_copy(h_hbm.at[2 * wid], V0)
    pltpu.sync_copy(h_hbm.at[2 * wid + 1], V1)

    def _issue(ch, b):
        pltpu.async_copy(ep_hbm.at[pl.ds(ch * CE, CE)], eb[b], sems[b])

    def _wait(b):
        pltpu.make_async_copy(ep_hbm.at[pl.ds(0, CE)], eb[b], sems[b]).wait()

    def _edge_pass(grp_fn):
        def _process(b):
            @plsc.parallel_loop(0, CE // 16, unroll=32)
            def _grp(g):
                grp_fn(eb[b], g)

        _issue(0, 0)

        def _pair(p, c):
            ch0 = p * 2
            _issue(ch0 + 1, 1)
            _wait(0)
            _process(0)

            @pl.when(p + 1 < NCHUNK // 2)
            def _():
                _issue(ch0 + 2, 0)
            _wait(1)
            _process(1)
            return c
        lax.fori_loop(0, NCHUNK // 2, _pair, 0)

    t0 = tv[pl.ds(0, 16)]

    @plsc.parallel_loop(0, NV, unroll=4)
    def _node_init(i):
        sl = pl.ds(i * 16, 16)
        y = dsv[sl]
        q0 = y * V0[sl]
        V0[sl] = q0
        T0[sl] = q0
        H0[sl] = t0 * q0
        q1 = y * V1[sl]
        V1[sl] = q1
        T1[sl] = q1
        H1[sl] = t0 * q1

    def _edge_grp(ebuf, g):
        ev = ebuf[pl.ds(g * 16, 16)]
        rv = jnp.bitwise_and(ev, jnp.int32(0xFFFF))
        cv = lax.shift_right_logical(ev, 16)
        g0 = plsc.load_gather(V0, [rv])
        g1 = plsc.load_gather(V1, [rv])
        plsc.addupdate_scatter(T0, [cv], g0)
        plsc.addupdate_scatter(T1, [cv], g1)

    for k in range(K):
        with jax.named_scope("edge_hop"):
            _edge_pass(_edge_grp)
        tk = tv[pl.ds(16 * (k + 1), 16)]

        @plsc.parallel_loop(0, NV, unroll=8)
        def _scale(i):
            sl = pl.ds(i * 16, 16)
            s = dsv[sl]
            ss = s * s
            q0 = ss * T0[sl]
            V0[sl] = q0
            T0[sl] = q0
            H0[sl] = H0[sl] + tk * q0
            q1 = ss * T1[sl]
            V1[sl] = q1
            T1[sl] = q1
            H1[sl] = H1[sl] + tk * q1

    @plsc.parallel_loop(0, NV, unroll=8)
    def _fin(i):
        sl = pl.ds(i * 16, 16)
        s = dsv[sl]
        V0[sl] = H0[sl] / s
        V1[sl] = H1[sl] / s
    pltpu.sync_copy(V0, out_hbm.at[2 * wid])
    pltpu.sync_copy(V1, out_hbm.at[2 * wid + 1])


_sc_prop = functools.partial(
    pl.kernel,
    out_type=jax.ShapeDtypeStruct((CP, N), jnp.float32),
    mesh=plsc.VectorSubcoreMesh(core_axis_name="c", subcore_axis_name="s"),
    compiler_params=pltpu.CompilerParams(needs_layout_passes=False,
                                         disable_bounds_checks=True),
    scratch_types=[
        pltpu.VMEM((N,), jnp.float32),
        pltpu.VMEM((N,), jnp.float32),
        pltpu.VMEM((N,), jnp.float32),
        pltpu.VMEM((N,), jnp.float32),
        pltpu.VMEM((N,), jnp.float32),
        pltpu.VMEM((N,), jnp.float32),
        pltpu.VMEM((N,), jnp.float32),
        pltpu.VMEM((CE,), jnp.int32),
        pltpu.VMEM((CE,), jnp.int32),
        pltpu.VMEM((16 * (K + 1),), jnp.float32),
        pltpu.SemaphoreType.DMA,
        pltpu.SemaphoreType.DMA,
    ],
)(_sc_body)


def kernel(x, edge_index, W1, b1, W2, b2, temp):
    w2p = jnp.pad(W2, ((0, 0), (0, CP - C)))
    b2p = jnp.pad(b2, (0, CP - C)).reshape(1, CP)
    ei = edge_index.astype(jnp.int32)
    ep = jnp.bitwise_or(ei[0], jnp.left_shift(ei[1], 16))

    degp = _sc_deg(ep)
    h_pad, dsv = pl.pallas_call(
        _mlp_body,
        out_shape=(jax.ShapeDtypeStruct((N, CP), jnp.float32),
                   jax.ShapeDtypeStruct((1, N), jnp.float32)),
    )(x, W1, b1.reshape(1, HID), w2p, b2p, degp)

    h64 = h_pad.transpose(1, 0)
    tb = jnp.broadcast_to(temp.reshape(K + 1, 1), (K + 1, 16)).reshape(-1)
    u64 = _sc_prop(h64, ep, tb, dsv.reshape(N))
    u = u64.transpose(1, 0)

    return pl.pallas_call(
        _lsm_body,
        out_shape=jax.ShapeDtypeStruct((N, C), jnp.float32),
    )(u)
